# Initial kernel scaffold; baseline (speedup 1.0000x reference)
import jax
import jax.numpy as jnp
from jax.experimental import pallas as pl


def kernel(user_idx, poi_embeddings, edge_index, user_table, Wp, bp, Wg1, bg1, Wg2, bg2, Wf, bf):
    raise NotImplementedError("write your pallas kernel here")


# speedup vs baseline: 22.6478x; 22.6478x over previous
import functools

import jax
import jax.numpy as jnp
from jax import lax
from jax.experimental import pallas as pl
from jax.experimental.pallas import tpu as pltpu
from jax.experimental.pallas import tpu_sc as plsc

NUM_USERS = 8000
NUM_POIS = 2000
N_NODES = 10000
N_PAD = 10240
DIM = 128
POI_DIM = 256
N_EDGES = 320000
BATCH = 4096

NC = 2
NS = 16
NW = NC * NS
LANES = 16

EPW = N_EDGES // NW
CHUNK = 125
NCHUNK = EPW // CHUNK
ROWS_PER_TILE = N_PAD // NS
OUT_STEP = 128
NOUT = ROWS_PER_TILE // OUT_STEP
BPW = BATCH // NW

_MESH = plsc.VectorSubcoreMesh(core_axis_name="c", subcore_axis_name="s")
_SC_PARAMS = pltpu.CompilerParams(needs_layout_passes=False)


@functools.partial(
    pl.kernel,
    out_type=jax.ShapeDtypeStruct((NW * N_NODES,), jnp.float32),
    mesh=_MESH,
    scratch_types=[
        pltpu.VMEM((EPW,), jnp.int32),
        pltpu.VMEM((N_NODES,), jnp.float32),
    ],
    compiler_params=_SC_PARAMS,
)
def _deg_kernel(dst_hbm, degp_hbm, idx_v, deg_v):
    c = lax.axis_index("c")
    s = lax.axis_index("s")
    wid = s * NC + c

    zeros16 = jnp.zeros((LANES,), jnp.float32)

    def zero_body(i, carry):
        deg_v[pl.ds(i * LANES, LANES)] = zeros16
        return carry

    lax.fori_loop(0, N_NODES // LANES, zero_body, 0)

    pltpu.sync_copy(dst_hbm.at[pl.ds(wid * EPW, EPW)], idx_v)

    ones16 = jnp.ones((LANES,), jnp.float32)

    def hist_body(i, carry):
        idx = idx_v[pl.ds(i * LANES, LANES)]
        plsc.addupdate_scatter(deg_v, [idx], ones16)
        return carry

    lax.fori_loop(0, EPW // LANES, hist_body, 0)

    pltpu.sync_copy(deg_v, degp_hbm.at[pl.ds(wid * N_NODES, N_NODES)])


@functools.partial(
    pl.kernel,
    out_type=(
        jax.ShapeDtypeStruct((N_PAD, DIM), jnp.float32),
        jax.ShapeDtypeStruct((N_PAD, DIM), jnp.float32),
    ),
    mesh=_MESH,
    scratch_types=[
        pltpu.VMEM((NCHUNK, CHUNK), jnp.int32),
        pltpu.VMEM((NCHUNK, CHUNK), jnp.int32),
        pltpu.VMEM((OUT_STEP, DIM), jnp.float32),
        pltpu.VMEM_SHARED((N_PAD, DIM), jnp.float32),
        pltpu.SemaphoreType.DMA,
    ],
    compiler_params=_SC_PARAMS,
)
def _agg_kernel(g_hbm, src_hbm, dst_hbm, zeros_hbm, out_a, out_b,
                sidx_v, didx_v, rows_v, acc_sh, sem):
    c = lax.axis_index("c")
    s = lax.axis_index("s")
    wid = s * NC + c

    pltpu.sync_copy(zeros_hbm, rows_v)

    def zero_body(t, carry):
        pltpu.sync_copy(
            rows_v,
            acc_sh.at[pl.ds(s * ROWS_PER_TILE + t * OUT_STEP, OUT_STEP)])
        return carry

    lax.fori_loop(0, NOUT, zero_body, 0)

    pltpu.sync_copy(src_hbm.at[wid], sidx_v)
    pltpu.sync_copy(dst_hbm.at[wid], didx_v)

    plsc.subcore_barrier()

    def edge_body(j, carry):
        crows = rows_v.at[pl.ds(0, CHUNK)]
        pltpu.async_copy(g_hbm.at[sidx_v.at[j]], crows, sem).wait()
        pltpu.sync_copy(crows, acc_sh.at[didx_v.at[j]], add=True)
        return carry

    lax.fori_loop(0, NCHUNK, edge_body, 0)

    plsc.subcore_barrier()

    def out_body(t, carry):
        r = s * ROWS_PER_TILE + t * OUT_STEP
        pltpu.sync_copy(acc_sh.at[pl.ds(r, OUT_STEP)], rows_v)

        @pl.when(c == 0)
        def _():
            pltpu.sync_copy(rows_v, out_a.at[pl.ds(r, OUT_STEP)])

        @pl.when(c == 1)
        def _():
            pltpu.sync_copy(rows_v, out_b.at[pl.ds(r, OUT_STEP)])

        return carry

    lax.fori_loop(0, NOUT, out_body, 0)


@functools.partial(
    pl.kernel,
    out_type=(
        jax.ShapeDtypeStruct((BATCH, DIM), jnp.float32),
        jax.ShapeDtypeStruct((BATCH, DIM), jnp.float32),
        jax.ShapeDtypeStruct((BATCH, DIM), jnp.float32),
    ),
    mesh=_MESH,
    scratch_types=[
        pltpu.VMEM((BPW,), jnp.int32),
        pltpu.VMEM((BPW, DIM), jnp.float32),
        pltpu.SemaphoreType.DMA,
    ],
    compiler_params=_SC_PARAMS,
)
def _gather_kernel(sa, sb, g2, ut, dinv128, uidx, y1, y2, y3,
                   uidx_v, rows_v, sem):
    c = lax.axis_index("c")
    s = lax.axis_index("s")
    wid = s * NC + c
    base = wid * BPW

    pltpu.sync_copy(uidx.at[pl.ds(base, BPW)], uidx_v)
    pltpu.async_copy(sa.at[uidx_v], rows_v, sem).wait()
    pltpu.async_copy(sb.at[uidx_v], rows_v, sem, add=True).wait()
    pltpu.async_copy(g2.at[uidx_v], rows_v, sem, add=True).wait()
    pltpu.sync_copy(rows_v, y1.at[pl.ds(base, BPW)])
    pltpu.async_copy(ut.at[uidx_v], rows_v, sem).wait()
    pltpu.sync_copy(rows_v, y2.at[pl.ds(base, BPW)])
    pltpu.async_copy(dinv128.at[uidx_v], rows_v, sem).wait()
    pltpu.sync_copy(rows_v, y3.at[pl.ds(base, BPW)])


def _dense1_body(ut_ref, poi_ref, Wp_ref, bp_ref, Wg1_ref, degp_ref,
                 g1_ref, dinv16_ref, dinv128_ref):
    deg = jnp.sum(degp_ref[...], axis=0) + 1.0
    dinv = lax.rsqrt(deg)
    hp = jnp.dot(poi_ref[...], Wp_ref[...],
                 preferred_element_type=jnp.float32) + bp_ref[...]
    h1u = jnp.dot(ut_ref[...], Wg1_ref[...],
                  preferred_element_type=jnp.float32)
    h1p = jnp.dot(hp, Wg1_ref[...], preferred_element_type=jnp.float32)
    g1_ref[0:NUM_USERS, :] = h1u * dinv[0:NUM_USERS, None]
    g1_ref[NUM_USERS:N_NODES, :] = h1p * dinv[NUM_USERS:N_NODES, None]
    g1_ref[N_NODES:N_PAD, :] = jnp.zeros((N_PAD - N_NODES, DIM), jnp.float32)
    dinv16_ref[0:N_NODES, :] = jnp.broadcast_to(dinv[:, None],
                                                (N_NODES, LANES))
    dinv16_ref[N_NODES:N_PAD, :] = jnp.ones((N_PAD - N_NODES, LANES),
                                            jnp.float32)
    dinv128_ref[0:N_NODES, :] = jnp.broadcast_to(dinv[:, None],
                                                 (N_NODES, DIM))
    dinv128_ref[N_NODES:N_PAD, :] = jnp.ones((N_PAD - N_NODES, DIM),
                                             jnp.float32)


_dense1 = pl.pallas_call(
    _dense1_body,
    out_shape=(
        jax.ShapeDtypeStruct((N_PAD, DIM), jnp.float32),
        jax.ShapeDtypeStruct((N_PAD, LANES), jnp.float32),
        jax.ShapeDtypeStruct((N_PAD, DIM), jnp.float32),
    ),
)


def _dense2_body(sa_ref, sb_ref, g1_ref, dinv16_ref, bg1_ref, Wg2_ref,
                 g2_ref):
    dinv = dinv16_ref[:, 0:1]
    t = dinv * (sa_ref[...] + sb_ref[...] + g1_ref[...]) + bg1_ref[...]
    x1 = jnp.maximum(t, 0.2 * t)
    g2_ref[...] = jnp.dot(x1, Wg2_ref[...],
                          preferred_element_type=jnp.float32) * dinv


_dense2 = pl.pallas_call(
    _dense2_body,
    out_shape=jax.ShapeDtypeStruct((N_PAD, DIM), jnp.float32),
)


def _final_body(y1_ref, y2_ref, y3_ref, bg2_ref, Wf_ref, bf_ref, out_ref):
    t = y3_ref[...] * y1_ref[...] + bg2_ref[...]
    x2 = jnp.maximum(t, 0.2 * t)
    out_ref[...] = jnp.dot(x2 + y2_ref[...], Wf_ref[...],
                           preferred_element_type=jnp.float32) + bf_ref[...]


_final = pl.pallas_call(
    _final_body,
    out_shape=jax.ShapeDtypeStruct((BATCH, DIM), jnp.float32),
)


@jax.jit
def kernel(user_idx, poi_embeddings, edge_index, user_table,
           Wp, bp, Wg1, bg1, Wg2, bg2, Wf, bf):
    src = edge_index[0].astype(jnp.int32)
    dst = edge_index[1].astype(jnp.int32)
    src3 = src.reshape(NW, NCHUNK, CHUNK)
    dst3 = dst.reshape(NW, NCHUNK, CHUNK)
    zeros = jnp.zeros((OUT_STEP, DIM), jnp.float32)
    uidx = user_idx.astype(jnp.int32)

    degp = _deg_kernel(dst).reshape(NW, N_NODES)
    g1, dinv16, dinv128 = _dense1(user_table, poi_embeddings, Wp, bp, Wg1,
                                  degp)
    s1a, s1b = _agg_kernel(g1, src3, dst3, zeros)
    g2 = _dense2(s1a, s1b, g1, dinv16, bg1, Wg2)
    s2a, s2b = _agg_kernel(g2, src3, dst3, zeros)
    y1, y2, y3 = _gather_kernel(s2a, s2b, g2, user_table, dinv128, uidx)
    return _final(y1, y2, y3, bg2, Wf, bf)
